# baseline (speedup 1.0000x reference)
import functools

import jax
import jax.numpy as jnp
from jax import lax
from jax.experimental import pallas as pl
from jax.experimental.pallas import tpu as pltpu
from jax.experimental.pallas import tpu_sc as plsc

B = 256
NP = 10240
HALF = NP // 2
ROWS = 2560
NEG = -1e30
IMAX = 2147483647

NW = 32
CH = HALF // NW
NCH = CH // 16
SEGT = 272


def _mlp_body(
    e_ref, w1_ref, b1_ref, w2_ref, b2_ref, w3_ref, b3_ref, u_ref, lg_ref, gum_ref
):
    eb = e_ref[...]
    h1 = jax.lax.dot_general(
        w1_ref[...], eb, (((0,), (1,)), ((), ())), preferred_element_type=jnp.float32
    )
    h1 = jnp.maximum(h1 + b1_ref[...], 0.0)
    h2 = jax.lax.dot_general(
        w2_ref[...], h1, (((0,), (0,)), ((), ())), preferred_element_type=jnp.float32
    )
    h2 = jnp.maximum(h2 + b2_ref[...], 0.0)
    lb = jax.lax.dot_general(
        w3_ref[...], h2, (((0,), (0,)), ((), ())), preferred_element_type=jnp.float32
    )
    lg_ref[...] = lb + b3_ref[...]
    gum_ref[...] = -jnp.log(-jnp.log(u_ref[...]))


def _make_sc_body(h0):
    def _sc_body(
        lg_hbm, seg_hbm, gum_hbm,
        m_out, s_out, am_out, ag_out,
        lg_v, seg_v, gum_v, m_v, s_v, am_v, ag_v,
    ):
        wid = lax.axis_index("s") * 2 + lax.axis_index("c")
        base = wid * CH
        pltpu.sync_copy(lg_hbm.at[pl.ds(base, CH)], lg_v)
        pltpu.sync_copy(seg_hbm.at[pl.ds(base, CH)], seg_v)
        pltpu.sync_copy(gum_hbm.at[pl.ds(base, CH)], gum_v)

        lanes = lax.broadcasted_iota(jnp.int32, (16,), 0)
        for t in range(SEGT // 16):
            m_v[pl.ds(t * 16, 16)] = jnp.full((16,), NEG, jnp.float32)
            s_v[pl.ds(t * 16, 16)] = jnp.zeros((16,), jnp.float32)
            am_v[pl.ds(t * 16, 16)] = jnp.full((16,), NEG, jnp.float32)
            ag_v[pl.ds(t * 16, 16)] = jnp.full((16,), IMAX, jnp.int32)

        def _gat(x, idx):
            return lax.gather(
                x,
                idx[:, None],
                lax.GatherDimensionNumbers(
                    offset_dims=(), collapsed_slice_dims=(0,), start_index_map=(0,)
                ),
                slice_sizes=(1,),
                mode=lax.GatherScatterMode.PROMISE_IN_BOUNDS,
            )

        def _shift(x, k):
            return _gat(x, jnp.maximum(lanes - k, 0))

        for j in range(NCH):
            off = j * 16
            v = lg_v[pl.ds(off, 16)]
            s = seg_v[pl.ds(off, 16)]
            g = gum_v[pl.ds(off, 16)]
            sv = v + g
            ix = lanes + (h0 + base + off)
            mv = v
            for k in (1, 2, 4, 8):
                same = (_shift(s, k) == s) & (lanes >= k)
                mv = jnp.where(same, jnp.maximum(mv, _shift(mv, k)), mv)
                psv = _shift(sv, k)
                pix = _shift(ix, k)
                take = same & ((psv > sv) | ((psv == sv) & (pix < ix)))
                sv = jnp.where(take, psv, sv)
                ix = jnp.where(take, pix, ix)
            nxt = _gat(s, jnp.minimum(lanes + 1, 15))
            slast = (nxt != s) | (lanes == 15)
            cur = plsc.load_gather(m_v, [s])
            plsc.store_scatter(m_v, [s], jnp.maximum(cur, mv), mask=slast)
            curv = plsc.load_gather(am_v, [s])
            curi = plsc.load_gather(ag_v, [s])
            better = (sv > curv) | ((sv == curv) & (ix < curi))
            plsc.store_scatter(am_v, [s], jnp.where(better, sv, curv), mask=slast)
            plsc.store_scatter(ag_v, [s], jnp.where(better, ix, curi), mask=slast)

        for j in range(NCH):
            off = j * 16
            v = lg_v[pl.ds(off, 16)]
            s = seg_v[pl.ds(off, 16)]
            mseg = plsc.load_gather(m_v, [s])
            acc = jnp.exp(v - mseg)
            for k in (1, 2, 4, 8):
                same = (_shift(s, k) == s) & (lanes >= k)
                acc = jnp.where(same, acc + _shift(acc, k), acc)
            nxt = _gat(s, jnp.minimum(lanes + 1, 15))
            slast = (nxt != s) | (lanes == 15)
            cur = plsc.load_gather(s_v, [s])
            plsc.store_scatter(s_v, [s], cur + acc, mask=slast)

        pltpu.sync_copy(m_v, m_out.at[wid])
        pltpu.sync_copy(s_v, s_out.at[wid])
        pltpu.sync_copy(am_v, am_out.at[wid])
        pltpu.sync_copy(ag_v, ag_out.at[wid])

    return _sc_body


def _finish_body(
    mp_ref, sp_ref, ap_ref, gp_ref, segf_ref, lg_ref, off_ref, lp_ref, act_ref
):
    mp = mp_ref[...]
    m = jnp.max(mp, axis=0, keepdims=True)
    s = jnp.sum(sp_ref[...] * jnp.exp(mp - m), axis=0, keepdims=True)
    ap = ap_ref[...]
    amax = jnp.max(ap, axis=0, keepdims=True)
    arg = jnp.min(
        jnp.where(ap == amax, gp_ref[...], jnp.int32(IMAX)), axis=0, keepdims=True
    )
    mlz = jnp.where(s > 0.0, m + jnp.log(s), 0.0)[:, :B]
    idsf = jax.lax.broadcasted_iota(jnp.int32, (B, NP), 0)
    maskf = (segf_ref[...] == idsf).astype(jnp.float32)
    mlz_hi = mlz.astype(jnp.bfloat16).astype(jnp.float32)
    mlz_lo = mlz - mlz_hi
    dn = (((1,), (0,)), ((), ()))
    mlz_node = jax.lax.dot_general(
        mlz_hi, maskf, dn, preferred_element_type=jnp.float32
    ) + jax.lax.dot_general(mlz_lo, maskf, dn, preferred_element_type=jnp.float32)
    lp_ref[...] = lg_ref[...] - mlz_node
    act_ref[...] = arg[:, :B] - off_ref[...]


def _mlp_half(e, W1, b1, W2, b2, W3, b3, u_p, half):
    k = e.shape[1]
    h = W1.shape[1]
    return pl.pallas_call(
        _mlp_body,
        grid=(HALF // ROWS,),
        in_specs=[
            pl.BlockSpec((ROWS, k), lambda i: (i + half * (HALF // ROWS), 0)),
            pl.BlockSpec((k, h), lambda i: (0, 0)),
            pl.BlockSpec((h, 1), lambda i: (0, 0)),
            pl.BlockSpec((h, h), lambda i: (0, 0)),
            pl.BlockSpec((h, 1), lambda i: (0, 0)),
            pl.BlockSpec((h, 1), lambda i: (0, 0)),
            pl.BlockSpec((1, 1), lambda i: (0, 0)),
            pl.BlockSpec((1, ROWS), lambda i: (0, i + half * (HALF // ROWS))),
        ],
        out_specs=[
            pl.BlockSpec((1, ROWS), lambda i: (0, i)),
            pl.BlockSpec((1, ROWS), lambda i: (0, i)),
        ],
        out_shape=[
            jax.ShapeDtypeStruct((1, HALF), jnp.float32),
            jax.ShapeDtypeStruct((1, HALF), jnp.float32),
        ],
    )(e, W1, b1.reshape(h, 1), W2, b2.reshape(h, 1), W3, b3.reshape(1, 1), u_p)


def _sc_half(lg_h, seg_h, gum_h, h0):
    fn = functools.partial(
        pl.kernel,
        mesh=plsc.VectorSubcoreMesh(core_axis_name="c", subcore_axis_name="s"),
        out_type=[
            jax.ShapeDtypeStruct((NW, SEGT), jnp.float32),
            jax.ShapeDtypeStruct((NW, SEGT), jnp.float32),
            jax.ShapeDtypeStruct((NW, SEGT), jnp.float32),
            jax.ShapeDtypeStruct((NW, SEGT), jnp.int32),
        ],
        scratch_types=[
            pltpu.VMEM((CH,), jnp.float32),
            pltpu.VMEM((CH,), jnp.int32),
            pltpu.VMEM((CH,), jnp.float32),
            pltpu.VMEM((SEGT,), jnp.float32),
            pltpu.VMEM((SEGT,), jnp.float32),
            pltpu.VMEM((SEGT,), jnp.float32),
            pltpu.VMEM((SEGT,), jnp.int32),
        ],
        compiler_params=pltpu.CompilerParams(needs_layout_passes=False),
    )(_make_sc_body(h0))
    return fn(lg_h, seg_h, gum_h)


def kernel(e, u, batch_non_omni, act_offsets, W1, b1, W2, b2, W3, b3):
    n = e.shape[0]
    pad = NP - n

    seg_p = jnp.concatenate(
        [batch_non_omni, jnp.full((pad,), B, jnp.int32)]
    ).reshape(1, NP)
    u_p = jnp.concatenate([u, jnp.full((pad,), 0.5, jnp.float32)]).reshape(1, NP)

    lg1, gum1 = _mlp_half(e, W1, b1, W2, b2, W3, b3, u_p, 0)
    lg2, gum2 = _mlp_half(e, W1, b1, W2, b2, W3, b3, u_p, 1)

    seg1 = seg_p[:, :HALF].reshape(HALF)
    seg2 = seg_p[:, HALF:].reshape(HALF)
    p1 = _sc_half(lg1.reshape(HALF), seg1, gum1.reshape(HALF), 0)
    p2 = _sc_half(lg2.reshape(HALF), seg2, gum2.reshape(HALF), HALF)
    mp, sp, ap, gp = (jnp.concatenate([a, b]) for a, b in zip(p1, p2))

    lg_p = jnp.concatenate([lg1, lg2], axis=1)

    lp_p, act2 = pl.pallas_call(
        _finish_body,
        in_specs=[
            pl.BlockSpec((2 * NW, SEGT), lambda: (0, 0)),
            pl.BlockSpec((2 * NW, SEGT), lambda: (0, 0)),
            pl.BlockSpec((2 * NW, SEGT), lambda: (0, 0)),
            pl.BlockSpec((2 * NW, SEGT), lambda: (0, 0)),
            pl.BlockSpec((1, NP), lambda: (0, 0)),
            pl.BlockSpec((1, NP), lambda: (0, 0)),
            pl.BlockSpec((1, B), lambda: (0, 0)),
        ],
        out_specs=[
            pl.BlockSpec((1, NP), lambda: (0, 0)),
            pl.BlockSpec((1, B), lambda: (0, 0)),
        ],
        out_shape=[
            jax.ShapeDtypeStruct((1, NP), jnp.float32),
            jax.ShapeDtypeStruct((1, B), jnp.int32),
        ],
    )(mp, sp, ap, gp, seg_p, lg_p, act_offsets.reshape(1, B))

    logits = lg_p.reshape(NP)[:n]
    log_probs = lp_p.reshape(NP)[:n]
    act = act2.reshape(B)
    return (logits, log_probs, act)


# speedup vs baseline: 1.1482x; 1.1482x over previous
import functools

import jax
import jax.numpy as jnp
from jax import lax
from jax.experimental import pallas as pl
from jax.experimental.pallas import tpu as pltpu
from jax.experimental.pallas import tpu_sc as plsc

B = 256
NP = 10240
ROWS = 2048
NEG = -1e30
IMAX = 2147483647

NW = 32
CH = NP // NW
NCH = CH // 16
SEGT = 272


def _mlp_body(
    e_ref, w1_ref, b1_ref, w2_ref, b2_ref, w3_ref, b3_ref, u_ref, lg_ref, gum_ref
):
    eb = e_ref[...]
    h1 = jax.lax.dot_general(
        w1_ref[...], eb, (((0,), (1,)), ((), ())), preferred_element_type=jnp.float32
    )
    h1 = jnp.maximum(h1 + b1_ref[...], 0.0)
    h2 = jax.lax.dot_general(
        w2_ref[...], h1, (((0,), (0,)), ((), ())), preferred_element_type=jnp.float32
    )
    h2 = jnp.maximum(h2 + b2_ref[...], 0.0)
    lb = jax.lax.dot_general(
        w3_ref[...], h2, (((0,), (0,)), ((), ())), preferred_element_type=jnp.float32
    )
    lg_ref[...] = lb + b3_ref[...]
    gum_ref[...] = -jnp.log(-jnp.log(u_ref[...]))


def _sc_body(
    lg_hbm, seg_hbm, gum_hbm,
    m_out, s_out, am_out, ag_out,
    lg_v, seg_v, gum_v, m_v, s_v, am_v, ag_v,
):
    wid = lax.axis_index("s") * 2 + lax.axis_index("c")
    base = wid * CH
    pltpu.sync_copy(lg_hbm.at[pl.ds(base, CH)], lg_v)
    pltpu.sync_copy(seg_hbm.at[pl.ds(base, CH)], seg_v)
    pltpu.sync_copy(gum_hbm.at[pl.ds(base, CH)], gum_v)

    lanes = lax.broadcasted_iota(jnp.int32, (16,), 0)
    for t in range(SEGT // 16):
        m_v[pl.ds(t * 16, 16)] = jnp.full((16,), NEG, jnp.float32)
        s_v[pl.ds(t * 16, 16)] = jnp.zeros((16,), jnp.float32)
        am_v[pl.ds(t * 16, 16)] = jnp.full((16,), NEG, jnp.float32)
        ag_v[pl.ds(t * 16, 16)] = jnp.full((16,), IMAX, jnp.int32)

    def _gat(x, idx):
        return lax.gather(
            x,
            idx[:, None],
            lax.GatherDimensionNumbers(
                offset_dims=(), collapsed_slice_dims=(0,), start_index_map=(0,)
            ),
            slice_sizes=(1,),
            mode=lax.GatherScatterMode.PROMISE_IN_BOUNDS,
        )

    def _shift(x, k):
        return _gat(x, jnp.maximum(lanes - k, 0))

    def pass_a(j, _):
        off = j * 16
        v = lg_v[pl.ds(off, 16)]
        s = seg_v[pl.ds(off, 16)]
        g = gum_v[pl.ds(off, 16)]
        sv = v + g
        ix = lanes + (base + off)
        mv = v
        for k in (1, 2, 4, 8):
            same = (_shift(s, k) == s) & (lanes >= k)
            mv = jnp.where(same, jnp.maximum(mv, _shift(mv, k)), mv)
            psv = _shift(sv, k)
            pix = _shift(ix, k)
            take = same & ((psv > sv) | ((psv == sv) & (pix < ix)))
            sv = jnp.where(take, psv, sv)
            ix = jnp.where(take, pix, ix)
        nxt = _gat(s, jnp.minimum(lanes + 1, 15))
        slast = (nxt != s) | (lanes == 15)
        cur = plsc.load_gather(m_v, [s])
        plsc.store_scatter(m_v, [s], jnp.maximum(cur, mv), mask=slast)
        curv = plsc.load_gather(am_v, [s])
        curi = plsc.load_gather(ag_v, [s])
        better = (sv > curv) | ((sv == curv) & (ix < curi))
        plsc.store_scatter(am_v, [s], jnp.where(better, sv, curv), mask=slast)
        plsc.store_scatter(ag_v, [s], jnp.where(better, ix, curi), mask=slast)
        return _

    for j in range(NCH):
        pass_a(j, None)

    def pass_b(j, _):
        off = j * 16
        v = lg_v[pl.ds(off, 16)]
        s = seg_v[pl.ds(off, 16)]
        mseg = plsc.load_gather(m_v, [s])
        acc = jnp.exp(v - mseg)
        for k in (1, 2, 4, 8):
            same = (_shift(s, k) == s) & (lanes >= k)
            acc = jnp.where(same, acc + _shift(acc, k), acc)
        nxt = _gat(s, jnp.minimum(lanes + 1, 15))
        slast = (nxt != s) | (lanes == 15)
        cur = plsc.load_gather(s_v, [s])
        plsc.store_scatter(s_v, [s], cur + acc, mask=slast)
        return _

    for j in range(NCH):
        pass_b(j, None)

    pltpu.sync_copy(m_v, m_out.at[wid])
    pltpu.sync_copy(s_v, s_out.at[wid])
    pltpu.sync_copy(am_v, am_out.at[wid])
    pltpu.sync_copy(ag_v, ag_out.at[wid])


def _finish_body(
    mp_ref, sp_ref, ap_ref, gp_ref, segf_ref, lg_ref, off_ref, lp_ref, act_ref
):
    mp = mp_ref[...]
    m = jnp.max(mp, axis=0, keepdims=True)
    s = jnp.sum(sp_ref[...] * jnp.exp(mp - m), axis=0, keepdims=True)
    ap = ap_ref[...]
    amax = jnp.max(ap, axis=0, keepdims=True)
    arg = jnp.min(
        jnp.where(ap == amax, gp_ref[...], jnp.int32(IMAX)), axis=0, keepdims=True
    )
    mlz = jnp.where(s > 0.0, m + jnp.log(s), 0.0)[:, :B]
    idsf = jax.lax.broadcasted_iota(jnp.int32, (B, NP), 0)
    maskf = (segf_ref[...] == idsf).astype(jnp.float32)
    mlz_hi = mlz.astype(jnp.bfloat16).astype(jnp.float32)
    mlz_lo = mlz - mlz_hi
    dn = (((1,), (0,)), ((), ()))
    mlz_node = jax.lax.dot_general(
        mlz_hi, maskf, dn, preferred_element_type=jnp.float32
    ) + jax.lax.dot_general(mlz_lo, maskf, dn, preferred_element_type=jnp.float32)
    lp_ref[...] = lg_ref[...] - mlz_node
    act_ref[...] = arg[:, :B] - off_ref[...]


def kernel(e, u, batch_non_omni, act_offsets, W1, b1, W2, b2, W3, b3):
    n, k = e.shape
    h = W1.shape[1]
    pad = NP - n

    seg_p = jnp.concatenate(
        [batch_non_omni, jnp.full((pad,), B, jnp.int32)]
    ).reshape(1, NP)
    u_p = jnp.concatenate([u, jnp.full((pad,), 0.5, jnp.float32)]).reshape(1, NP)

    lg_p, gum_p = pl.pallas_call(
        _mlp_body,
        grid=(NP // ROWS,),
        in_specs=[
            pl.BlockSpec((ROWS, k), lambda i: (i, 0)),
            pl.BlockSpec((k, h), lambda i: (0, 0)),
            pl.BlockSpec((h, 1), lambda i: (0, 0)),
            pl.BlockSpec((h, h), lambda i: (0, 0)),
            pl.BlockSpec((h, 1), lambda i: (0, 0)),
            pl.BlockSpec((h, 1), lambda i: (0, 0)),
            pl.BlockSpec((1, 1), lambda i: (0, 0)),
            pl.BlockSpec((1, ROWS), lambda i: (0, i)),
        ],
        out_specs=[
            pl.BlockSpec((1, ROWS), lambda i: (0, i)),
            pl.BlockSpec((1, ROWS), lambda i: (0, i)),
        ],
        out_shape=[
            jax.ShapeDtypeStruct((1, NP), jnp.float32),
            jax.ShapeDtypeStruct((1, NP), jnp.float32),
        ],
    )(
        e,
        W1,
        b1.reshape(h, 1),
        W2,
        b2.reshape(h, 1),
        W3,
        b3.reshape(1, 1),
        u_p,
    )

    sc_fn = functools.partial(
        pl.kernel,
        mesh=plsc.VectorSubcoreMesh(core_axis_name="c", subcore_axis_name="s"),
        out_type=[
            jax.ShapeDtypeStruct((NW, SEGT), jnp.float32),
            jax.ShapeDtypeStruct((NW, SEGT), jnp.float32),
            jax.ShapeDtypeStruct((NW, SEGT), jnp.float32),
            jax.ShapeDtypeStruct((NW, SEGT), jnp.int32),
        ],
        scratch_types=[
            pltpu.VMEM((CH,), jnp.float32),
            pltpu.VMEM((CH,), jnp.int32),
            pltpu.VMEM((CH,), jnp.float32),
            pltpu.VMEM((SEGT,), jnp.float32),
            pltpu.VMEM((SEGT,), jnp.float32),
            pltpu.VMEM((SEGT,), jnp.float32),
            pltpu.VMEM((SEGT,), jnp.int32),
        ],
        compiler_params=pltpu.CompilerParams(needs_layout_passes=False),
    )(_sc_body)
    mp, sp, ap, gp = sc_fn(
        lg_p.reshape(NP), seg_p.reshape(NP), gum_p.reshape(NP)
    )

    lp_p, act2 = pl.pallas_call(
        _finish_body,
        in_specs=[
            pl.BlockSpec((NW, SEGT), lambda: (0, 0)),
            pl.BlockSpec((NW, SEGT), lambda: (0, 0)),
            pl.BlockSpec((NW, SEGT), lambda: (0, 0)),
            pl.BlockSpec((NW, SEGT), lambda: (0, 0)),
            pl.BlockSpec((1, NP), lambda: (0, 0)),
            pl.BlockSpec((1, NP), lambda: (0, 0)),
            pl.BlockSpec((1, B), lambda: (0, 0)),
        ],
        out_specs=[
            pl.BlockSpec((1, NP), lambda: (0, 0)),
            pl.BlockSpec((1, B), lambda: (0, 0)),
        ],
        out_shape=[
            jax.ShapeDtypeStruct((1, NP), jnp.float32),
            jax.ShapeDtypeStruct((1, B), jnp.int32),
        ],
    )(mp, sp, ap, gp, seg_p, lg_p, act_offsets.reshape(1, B))

    logits = lg_p.reshape(NP)[:n]
    log_probs = lp_p.reshape(NP)[:n]
    act = act2.reshape(B)
    return (logits, log_probs, act)
